# baseline (speedup 1.0000x reference)
import jax
import jax.numpy as jnp
from jax.experimental import pallas as pl

_BLK = 512


def _wide_kernel(x_ref, w_ref, b_ref, o_ref):
    o_ref[...] = (
        jnp.dot(x_ref[...], w_ref[...], preferred_element_type=jnp.float32)
        + b_ref[...]
    )


def kernel(feat, server_model, len_seq, mask, manfeat, emb1_w, emb2_w, emb3_w,
           emb4_w, emb5_w, k_w, o_w, cls_w, cls_b, wide_w, wide_b):
    b, k = manfeat.shape
    n = wide_w.shape[1]
    bias = wide_b.reshape(1, n)
    grid = (b // _BLK,)
    return pl.pallas_call(
        _wide_kernel,
        grid=grid,
        in_specs=[
            pl.BlockSpec((_BLK, k), lambda i: (i, 0)),
            pl.BlockSpec((k, n), lambda i: (0, 0)),
            pl.BlockSpec((1, n), lambda i: (0, 0)),
        ],
        out_specs=pl.BlockSpec((_BLK, n), lambda i: (i, 0)),
        out_shape=jax.ShapeDtypeStruct((b, n), jnp.float32),
    )(manfeat, wide_w, bias)


# speedup vs baseline: 1.2013x; 1.2013x over previous
import jax
import jax.numpy as jnp
from jax.experimental import pallas as pl

_BLK = 4096


def _wide_kernel(x_ref, w_ref, b_ref, o_ref):
    o_ref[...] = (
        jnp.dot(x_ref[...], w_ref[...], preferred_element_type=jnp.float32)
        + b_ref[...]
    )


def kernel(feat, server_model, len_seq, mask, manfeat, emb1_w, emb2_w, emb3_w,
           emb4_w, emb5_w, k_w, o_w, cls_w, cls_b, wide_w, wide_b):
    b, k = manfeat.shape
    n = wide_w.shape[1]
    bias = wide_b.reshape(1, n)
    grid = (b // _BLK,)
    return pl.pallas_call(
        _wide_kernel,
        grid=grid,
        in_specs=[
            pl.BlockSpec((_BLK, k), lambda i: (i, 0)),
            pl.BlockSpec((k, n), lambda i: (0, 0)),
            pl.BlockSpec((1, n), lambda i: (0, 0)),
        ],
        out_specs=pl.BlockSpec((_BLK, n), lambda i: (i, 0)),
        out_shape=jax.ShapeDtypeStruct((b, n), jnp.float32),
    )(manfeat, wide_w, bias)
